# Initial kernel scaffold; baseline (speedup 1.0000x reference)
import jax
import jax.numpy as jnp
from jax.experimental import pallas as pl


def kernel(action_states, Wq, bq, Wk, bk, log_temperature):
    raise NotImplementedError("write your pallas kernel here")


# speedup vs baseline: 40.4480x; 40.4480x over previous
import jax
import jax.numpy as jnp
import numpy as np
from jax.experimental import pallas as pl
from jax.experimental.pallas import tpu as pltpu

_B, _N, _D = 8, 2048, 256
_P = _D // 4
_TOPK = 8
_TR = 256


def _graph_kernel(lt_ref, a_full_ref, a_tile_ref, wq_ref, bq_ref, wk_ref,
                  bk_ref, out_ref, k_scratch):
    i = pl.program_id(1)

    @pl.when(i == 0)
    def _():
        a = a_full_ref[0]
        k = jax.lax.dot_general(a, wk_ref[...],
                                (((1,), (1,)), ((), ())),
                                preferred_element_type=jnp.float32)
        k_scratch[...] = k + bk_ref[...]

    temperature = jnp.clip(jnp.exp(lt_ref[0, 0]), 0.1, 10.0)
    inv_scale = 1.0 / (np.sqrt(float(_P)) * temperature)

    a_tile = a_tile_ref[0]
    q = jax.lax.dot_general(a_tile, wq_ref[...],
                            (((1,), (1,)), ((), ())),
                            preferred_element_type=jnp.float32)
    q = q + bq_ref[...]

    s = jax.lax.dot_general(q, k_scratch[...],
                            (((1,), (1,)), ((), ())),
                            preferred_element_type=jnp.float32)
    s = s * inv_scale

    m1 = jnp.max(s, axis=-1, keepdims=True)
    s_work = jnp.where(s == m1, -jnp.inf, s)
    for _ in range(_TOPK - 2):
        m = jnp.max(s_work, axis=-1, keepdims=True)
        s_work = jnp.where(s_work == m, -jnp.inf, s_work)
    thr = jnp.max(s_work, axis=-1, keepdims=True)

    keep = s >= thr
    e = jnp.where(keep, jnp.exp(s - m1), 0.0)
    denom = jnp.sum(e, axis=-1, keepdims=True)
    out_ref[0] = e / denom


def kernel(action_states, Wq, bq, Wk, bk, log_temperature):
    lt = jnp.reshape(log_temperature, (1, 1))
    bq2 = jnp.reshape(bq, (1, _P))
    bk2 = jnp.reshape(bk, (1, _P))

    return pl.pallas_call(
        _graph_kernel,
        grid=(_B, _N // _TR),
        in_specs=[
            pl.BlockSpec((1, 1), lambda b, i: (0, 0)),
            pl.BlockSpec((1, _N, _D), lambda b, i: (b, 0, 0)),
            pl.BlockSpec((1, _TR, _D), lambda b, i: (b, i, 0)),
            pl.BlockSpec((_P, _D), lambda b, i: (0, 0)),
            pl.BlockSpec((1, _P), lambda b, i: (0, 0)),
            pl.BlockSpec((_P, _D), lambda b, i: (0, 0)),
            pl.BlockSpec((1, _P), lambda b, i: (0, 0)),
        ],
        out_specs=pl.BlockSpec((1, _TR, _N), lambda b, i: (b, i, 0)),
        out_shape=jax.ShapeDtypeStruct((_B, _N, _N), jnp.float32),
        scratch_shapes=[pltpu.VMEM((_N, _P), jnp.float32)],
    )(lt, action_states, action_states, Wq, bq2, Wk, bk2)
